# baseline (speedup 1.0000x reference)
import functools

import jax
import jax.numpy as jnp
from jax import lax
from jax.experimental import pallas as pl
from jax.experimental.pallas import tpu as pltpu
from jax.experimental.pallas import tpu_sc as plsc

_B = 8
_N = 16384
_G = 512
_K = 32
_QB = 8
_NBLK = _B * _G // _QB
_BIG_I = 2 ** 30


def _fps_body(x_ref, y_ref, z_ref, qx_ref, qy_ref, qz_ref, fg_ref, dist_ref):
    x = x_ref[...]
    y = y_ref[...]
    z = z_ref[...]
    iota = lax.broadcasted_iota(jnp.int32, (_B, _N), 1)
    boff = lax.broadcasted_iota(jnp.int32, (_B, 1), 0) * _N
    giota = lax.broadcasted_iota(jnp.int32, (_B, _G), 1)
    dist_ref[...] = jnp.full((_B, _N), 1e10, jnp.float32)

    def body(i, carry):
        far, qxa, qya, qza, fga = carry
        sel = iota == far
        cx = jnp.sum(jnp.where(sel, x, 0.0), axis=1, keepdims=True)
        cy = jnp.sum(jnp.where(sel, y, 0.0), axis=1, keepdims=True)
        cz = jnp.sum(jnp.where(sel, z, 0.0), axis=1, keepdims=True)
        slot = giota == i
        qxa = jnp.where(slot, cx, qxa)
        qya = jnp.where(slot, cy, qya)
        qza = jnp.where(slot, cz, qza)
        fga = jnp.where(slot, far + boff, fga)
        dx = x - cx
        dy = y - cy
        dz = z - cz
        d = (dx * dx + dy * dy) + dz * dz
        dmin = jnp.minimum(dist_ref[...], d)
        dist_ref[...] = dmin
        m = jnp.max(dmin, axis=1, keepdims=True)
        cand = jnp.where(dmin == m, iota, _BIG_I)
        far = jnp.min(cand, axis=1, keepdims=True)
        return (far, qxa, qya, qza, fga)

    zf = jnp.zeros((_B, _G), jnp.float32)
    _, qxa, qya, qza, fga = lax.fori_loop(
        0, _G, body,
        (jnp.zeros((_B, 1), jnp.int32), zf, zf, zf,
         jnp.zeros((_B, _G), jnp.int32)))
    qx_ref[...] = qxa
    qy_ref[...] = qya
    qz_ref[...] = qza
    fg_ref[...] = fga


def _knn_body(x_ref, y_ref, z_ref, p8_ref, qx_ref, qy_ref, qz_ref,
              out_ref, d_ref, s_ref, a_ref, s2_ref, a2_ref, l_ref, gl_ref):
    p = pl.program_id(0)
    b = p // (_G // _QB)
    x = x_ref[0]
    y = y_ref[0]
    z = z_ref[0]
    qx = qx_ref[0]
    qy = qy_ref[0]
    qz = qz_ref[0]
    pn = (x * x + y * y) + z * z
    qn = (qx * qx + qy * qy) + qz * qz
    qpad = jnp.concatenate(
        [qx, qy, qz, jnp.zeros((_QB, 5), jnp.float32)], axis=1)
    mm = lax.dot_general(qpad, p8_ref[0], (((1,), (0,)), ((), ())),
                         precision=lax.Precision.DEFAULT)
    d_ref[...] = (-2.0 * mm + qn) + pn
    kiota = lax.broadcasted_iota(jnp.int32, (_QB, _K), 1)
    liota = lax.broadcasted_iota(jnp.int32, (_QB, 128), 1)
    inf = jnp.inf
    nv = _N // 128

    s = d_ref[:, 0:128]
    a = jnp.zeros((_QB, 128), jnp.int32)
    s2 = jnp.full((_QB, 128), inf, jnp.float32)
    a2 = jnp.zeros((_QB, 128), jnp.int32)
    for v in range(1, nv):
        dv = d_ref[:, v * 128:(v + 1) * 128]
        lt1 = dv < s
        loser = jnp.where(lt1, s, dv)
        losera = jnp.where(lt1, a, v)
        lt2 = loser < s2
        s2 = jnp.where(lt2, loser, s2)
        a2 = jnp.where(lt2, losera, a2)
        s = jnp.where(lt1, dv, s)
        a = jnp.where(lt1, v, a)
    s_ref[...] = s
    a_ref[...] = a
    s2_ref[...] = s2
    a2_ref[...] = a2
    l_ref[...] = jnp.full((_QB, 128), -inf, jnp.float32)
    gl_ref[...] = jnp.full((_QB, 128), -1, jnp.int32)

    def body(j, _):
        s = s_ref[...]
        a = a_ref[...]
        m = jnp.min(s, axis=1, keepdims=True)
        g = a * 128 + liota
        c = jnp.where(s == m, g, _BIG_I)
        gsel = jnp.min(c, axis=1, keepdims=True)
        out_ref[0] = jnp.where(kiota == j, gsel + b * _N, out_ref[0])
        hit = c == gsel
        l_ref[...] = jnp.where(hit, m, l_ref[...])
        gl_ref[...] = jnp.where(hit, gsel, gl_ref[...])
        s2 = s2_ref[...]
        s_new = jnp.where(hit, s2, s)
        s_ref[...] = s_new
        a_ref[...] = jnp.where(hit, a2_ref[...], a)
        s2_ref[...] = jnp.where(hit, inf, s2)

        @pl.when(jnp.any(s_new == inf))
        def _rescan():
            ll = l_ref[...]
            gl = gl_ref[...]
            ns = jnp.full((_QB, 128), inf, jnp.float32)
            na = jnp.zeros((_QB, 128), jnp.int32)
            ns2 = jnp.full((_QB, 128), inf, jnp.float32)
            na2 = jnp.zeros((_QB, 128), jnp.int32)
            for v in range(nv):
                dv = d_ref[:, v * 128:(v + 1) * 128]
                gv = v * 128 + liota
                ok = (dv > ll) | ((dv == ll) & (gv > gl))
                dd = jnp.where(ok, dv, inf)
                lt1 = dd < ns
                loser = jnp.where(lt1, ns, dd)
                losera = jnp.where(lt1, na, v)
                lt2 = loser < ns2
                ns2 = jnp.where(lt2, loser, ns2)
                na2 = jnp.where(lt2, losera, na2)
                ns = jnp.where(lt1, dd, ns)
                na = jnp.where(lt1, v, na)
            rr = s_ref[...] == inf
            s_ref[...] = jnp.where(rr, ns, s_ref[...])
            a_ref[...] = jnp.where(rr, na, a_ref[...])
            s2_ref[...] = jnp.where(rr, ns2, s2_ref[...])
            a2_ref[...] = jnp.where(rr, na2, a2_ref[...])

        return 0

    lax.fori_loop(0, _K, body, 0)


def _run_fps(x, y, z):
    return pl.pallas_call(
        _fps_body,
        out_shape=[
            jax.ShapeDtypeStruct((_B, _G), jnp.float32),
            jax.ShapeDtypeStruct((_B, _G), jnp.float32),
            jax.ShapeDtypeStruct((_B, _G), jnp.float32),
            jax.ShapeDtypeStruct((_B, _G), jnp.int32),
        ],
        scratch_shapes=[pltpu.VMEM((_B, _N), jnp.float32)],
    )(x, y, z)


def _run_knn(x3, y3, z3, p8, qxr, qyr, qzr):
    qspec = pl.BlockSpec((1, _QB, 1), lambda p: (p, 0, 0))
    xspec = pl.BlockSpec((1, 1, _N), lambda p: (p // (_G // _QB), 0, 0))
    pspec = pl.BlockSpec((1, 8, _N), lambda p: (p // (_G // _QB), 0, 0))
    return pl.pallas_call(
        _knn_body,
        grid=(_NBLK,),
        in_specs=[xspec, xspec, xspec, pspec, qspec, qspec, qspec],
        out_specs=pl.BlockSpec((1, _QB, _K), lambda p: (p, 0, 0)),
        out_shape=jax.ShapeDtypeStruct((_NBLK, _QB, _K), jnp.int32),
        scratch_shapes=[
            pltpu.VMEM((_QB, _N), jnp.float32),
            pltpu.VMEM((_QB, 128), jnp.float32),
            pltpu.VMEM((_QB, 128), jnp.int32),
            pltpu.VMEM((_QB, 128), jnp.float32),
            pltpu.VMEM((_QB, 128), jnp.int32),
            pltpu.VMEM((_QB, 128), jnp.float32),
            pltpu.VMEM((_QB, 128), jnp.int32),
        ],
    )(x3, y3, z3, p8, qxr, qyr, qzr)


_NW = 32
_CHUNK = 128


def _wid():
    return lax.axis_index("s") * 2 + lax.axis_index("c")


@functools.cache
def _sc_mesh():
    return plsc.VectorSubcoreMesh(core_axis_name="c", subcore_axis_name="s")


@functools.cache
def _gather_rows(n_rows):
    per_w = n_rows // _NW

    @functools.partial(
        pl.kernel,
        out_type=jax.ShapeDtypeStruct((n_rows, 128), jnp.float32),
        mesh=_sc_mesh(),
        scratch_types=[
            pltpu.VMEM((_CHUNK,), jnp.int32),
            pltpu.VMEM((_CHUNK, 128), jnp.float32),
            pltpu.SemaphoreType.DMA,
        ],
    )
    def run(tab_hbm, idx_hbm, out_hbm, idx_v, rows_v, sem):
        base = _wid() * per_w

        def chunk(i, _):
            off = base + i * _CHUNK
            pltpu.sync_copy(idx_hbm.at[pl.ds(off, _CHUNK)], idx_v)
            pltpu.async_copy(tab_hbm.at[idx_v], rows_v, sem).wait()
            pltpu.sync_copy(rows_v, out_hbm.at[pl.ds(off, _CHUNK)])
            return 0

        lax.fori_loop(0, per_w // _CHUNK, chunk, 0)

    return run


def kernel(xyz, points):
    x = xyz[:, :, 0]
    y = xyz[:, :, 1]
    z = xyz[:, :, 2]
    qx, qy, qz, fpsg = _run_fps(x, y, z)

    qxr = qx.reshape(_NBLK, _QB, 1)
    qyr = qy.reshape(_NBLK, _QB, 1)
    qzr = qz.reshape(_NBLK, _QB, 1)
    x3 = x.reshape(_B, 1, _N)
    y3 = y.reshape(_B, 1, _N)
    z3 = z.reshape(_B, 1, _N)
    p8 = jnp.concatenate([xyz.transpose(0, 2, 1),
                          jnp.zeros((_B, 5, _N), jnp.float32)], axis=1)
    knng = _run_knn(x3, y3, z3, p8, qxr, qyr, qzr)

    table = jnp.pad(jnp.concatenate([points, xyz], axis=-1),
                    ((0, 0), (0, 0), (0, 61))).reshape(_B * _N, 128)

    new_rows = _gather_rows(_B * _G)(table, fpsg.reshape(-1))
    grouped_rows = _gather_rows(_B * _G * _K)(table, knng.reshape(-1))

    new_xyz = jnp.stack([qx, qy, qz], axis=-1)
    new_points = new_rows[:, :64].reshape(_B, _G, 64)
    grouped_xyz = grouped_rows[:, 64:67].reshape(_B, _G, _K, 3)
    grouped_points = grouped_rows[:, :64].reshape(_B, _G, _K, 64)
    return (new_xyz, new_points, grouped_xyz, grouped_points)


# speedup vs baseline: 1.3605x; 1.3605x over previous
import functools

import jax
import jax.numpy as jnp
from jax import lax
from jax.experimental import pallas as pl
from jax.experimental.pallas import tpu as pltpu
from jax.experimental.pallas import tpu_sc as plsc

_B = 8
_N = 16384
_G = 512
_K = 32
_QB = 8
_NBLK = _B * _G // _QB
_BIG_I = 2 ** 30


def _fps_body(x_ref, y_ref, z_ref, qx_ref, qy_ref, qz_ref, fg_ref, dist_ref):
    x = x_ref[...]
    y = y_ref[...]
    z = z_ref[...]
    iota = lax.broadcasted_iota(jnp.int32, (_B, _N), 1)
    boff = lax.broadcasted_iota(jnp.int32, (_B, 1), 0) * _N
    giota = lax.broadcasted_iota(jnp.int32, (_B, _G), 1)
    dist_ref[...] = jnp.full((_B, _N), 1e10, jnp.float32)

    def body(i, carry):
        far, qxa, qya, qza, fga = carry
        sel = iota == far
        cx = jnp.sum(jnp.where(sel, x, 0.0), axis=1, keepdims=True)
        cy = jnp.sum(jnp.where(sel, y, 0.0), axis=1, keepdims=True)
        cz = jnp.sum(jnp.where(sel, z, 0.0), axis=1, keepdims=True)
        slot = giota == i
        qxa = jnp.where(slot, cx, qxa)
        qya = jnp.where(slot, cy, qya)
        qza = jnp.where(slot, cz, qza)
        fga = jnp.where(slot, far + boff, fga)
        dx = x - cx
        dy = y - cy
        dz = z - cz
        d = (dx * dx + dy * dy) + dz * dz
        dmin = jnp.minimum(dist_ref[...], d)
        dist_ref[...] = dmin
        m = jnp.max(dmin, axis=1, keepdims=True)
        cand = jnp.where(dmin == m, iota, _BIG_I)
        far = jnp.min(cand, axis=1, keepdims=True)
        return (far, qxa, qya, qza, fga)

    zf = jnp.zeros((_B, _G), jnp.float32)
    _, qxa, qya, qza, fga = lax.fori_loop(
        0, _G, body,
        (jnp.zeros((_B, 1), jnp.int32), zf, zf, zf,
         jnp.zeros((_B, _G), jnp.int32)))
    qx_ref[...] = qxa
    qy_ref[...] = qya
    qz_ref[...] = qza
    fg_ref[...] = fga


def _knn_body(x_ref, y_ref, z_ref, p8_ref, qx_ref, qy_ref, qz_ref,
              out_ref, d_ref, sr1, sr2, sr3, sr4, ar1, ar2, ar3, ar4,
              l_ref, gl_ref):
    s_refs = [sr1, sr2, sr3, sr4]
    a_refs = [ar1, ar2, ar3, ar4]
    p = pl.program_id(0)
    b = p // (_G // _QB)
    x = x_ref[0]
    y = y_ref[0]
    z = z_ref[0]
    qx = qx_ref[0]
    qy = qy_ref[0]
    qz = qz_ref[0]
    pn = (x * x + y * y) + z * z
    qn = (qx * qx + qy * qy) + qz * qz
    qpad = jnp.concatenate(
        [qx, qy, qz, jnp.zeros((_QB, 5), jnp.float32)], axis=1)
    mm = lax.dot_general(qpad, p8_ref[0], (((1,), (0,)), ((), ())),
                         precision=lax.Precision.DEFAULT)
    d_ref[...] = (-2.0 * mm + qn) + pn
    kiota = lax.broadcasted_iota(jnp.int32, (_QB, _K), 1)
    liota = lax.broadcasted_iota(jnp.int32, (_QB, 128), 1)
    inf = jnp.inf
    nv = _N // 128

    def fold4(levels, dv, va, filt=None):
        (s1, a1), (s2, a2), (s3, a3), (s4, a4) = levels
        lt1 = dv < s1
        l1 = jnp.where(lt1, s1, dv)
        la1 = jnp.where(lt1, a1, va)
        s1 = jnp.where(lt1, dv, s1)
        a1 = jnp.where(lt1, va, a1)
        lt2 = l1 < s2
        l2 = jnp.where(lt2, s2, l1)
        la2 = jnp.where(lt2, a2, la1)
        s2 = jnp.where(lt2, l1, s2)
        a2 = jnp.where(lt2, la1, a2)
        lt3 = l2 < s3
        l3 = jnp.where(lt3, s3, l2)
        la3 = jnp.where(lt3, a3, la2)
        s3 = jnp.where(lt3, l2, s3)
        a3 = jnp.where(lt3, la2, a3)
        lt4 = l3 < s4
        s4 = jnp.where(lt4, l3, s4)
        a4 = jnp.where(lt4, la3, a4)
        return [(s1, a1), (s2, a2), (s3, a3), (s4, a4)]

    zi = jnp.zeros((_QB, 128), jnp.int32)
    fi = jnp.full((_QB, 128), inf, jnp.float32)
    levels = [(fi, zi), (fi, zi), (fi, zi), (fi, zi)]
    for v in range(nv):
        levels = fold4(levels, d_ref[:, v * 128:(v + 1) * 128], v)
    for i in range(4):
        s_refs[i][...] = levels[i][0]
        a_refs[i][...] = levels[i][1]
    l_ref[...] = jnp.full((_QB, 128), -inf, jnp.float32)
    gl_ref[...] = jnp.full((_QB, 128), -1, jnp.int32)

    def body(j, _):
        s1 = s_refs[0][...]
        a1 = a_refs[0][...]
        m = jnp.min(s1, axis=1, keepdims=True)
        g = a1 * 128 + liota
        c = jnp.where(s1 == m, g, _BIG_I)
        gsel = jnp.min(c, axis=1, keepdims=True)
        out_ref[0] = jnp.where(kiota == j, gsel + b * _N, out_ref[0])
        hit = c == gsel
        l_ref[...] = jnp.where(hit, m, l_ref[...])
        gl_ref[...] = jnp.where(hit, gsel, gl_ref[...])
        s_new = jnp.where(hit, s_refs[1][...], s1)
        s_refs[0][...] = s_new
        a_refs[0][...] = jnp.where(hit, a_refs[1][...], a1)
        s_refs[1][...] = jnp.where(hit, s_refs[2][...], s_refs[1][...])
        a_refs[1][...] = jnp.where(hit, a_refs[2][...], a_refs[1][...])
        s_refs[2][...] = jnp.where(hit, s_refs[3][...], s_refs[2][...])
        a_refs[2][...] = jnp.where(hit, a_refs[3][...], a_refs[2][...])
        s_refs[3][...] = jnp.where(hit, inf, s_refs[3][...])

        @pl.when(jnp.any(s_new == inf))
        def _rescan():
            ll = l_ref[...]
            gl = gl_ref[...]
            nlv = [(fi, zi), (fi, zi), (fi, zi), (fi, zi)]
            for v in range(nv):
                dv = d_ref[:, v * 128:(v + 1) * 128]
                gv = v * 128 + liota
                ok = (dv > ll) | ((dv == ll) & (gv > gl))
                nlv = fold4(nlv, jnp.where(ok, dv, inf), v)
            rr = s_refs[0][...] == inf
            for i in range(4):
                s_refs[i][...] = jnp.where(rr, nlv[i][0], s_refs[i][...])
                a_refs[i][...] = jnp.where(rr, nlv[i][1], a_refs[i][...])

        return 0

    lax.fori_loop(0, _K, body, 0)


def _run_fps(x, y, z):
    return pl.pallas_call(
        _fps_body,
        out_shape=[
            jax.ShapeDtypeStruct((_B, _G), jnp.float32),
            jax.ShapeDtypeStruct((_B, _G), jnp.float32),
            jax.ShapeDtypeStruct((_B, _G), jnp.float32),
            jax.ShapeDtypeStruct((_B, _G), jnp.int32),
        ],
        scratch_shapes=[pltpu.VMEM((_B, _N), jnp.float32)],
    )(x, y, z)


def _run_knn(x3, y3, z3, p8, qxr, qyr, qzr):
    qspec = pl.BlockSpec((1, _QB, 1), lambda p: (p, 0, 0))
    xspec = pl.BlockSpec((1, 1, _N), lambda p: (p // (_G // _QB), 0, 0))
    pspec = pl.BlockSpec((1, 8, _N), lambda p: (p // (_G // _QB), 0, 0))
    return pl.pallas_call(
        _knn_body,
        grid=(_NBLK,),
        in_specs=[xspec, xspec, xspec, pspec, qspec, qspec, qspec],
        out_specs=pl.BlockSpec((1, _QB, _K), lambda p: (p, 0, 0)),
        out_shape=jax.ShapeDtypeStruct((_NBLK, _QB, _K), jnp.int32),
        scratch_shapes=(
            [pltpu.VMEM((_QB, _N), jnp.float32)]
            + [pltpu.VMEM((_QB, 128), jnp.float32)] * 4
            + [pltpu.VMEM((_QB, 128), jnp.int32)] * 4
            + [pltpu.VMEM((_QB, 128), jnp.float32),
               pltpu.VMEM((_QB, 128), jnp.int32)]
        ),
    )(x3, y3, z3, p8, qxr, qyr, qzr)


_NW = 32
_CHUNK = 128


def _wid():
    return lax.axis_index("s") * 2 + lax.axis_index("c")


@functools.cache
def _sc_mesh():
    return plsc.VectorSubcoreMesh(core_axis_name="c", subcore_axis_name="s")


@functools.cache
def _gather_rows(n_rows):
    per_w = n_rows // _NW

    @functools.partial(
        pl.kernel,
        out_type=jax.ShapeDtypeStruct((n_rows, 128), jnp.float32),
        mesh=_sc_mesh(),
        scratch_types=[
            pltpu.VMEM((_CHUNK,), jnp.int32),
            pltpu.VMEM((_CHUNK, 128), jnp.float32),
            pltpu.SemaphoreType.DMA,
        ],
    )
    def run(tab_hbm, idx_hbm, out_hbm, idx_v, rows_v, sem):
        base = _wid() * per_w

        def chunk(i, _):
            off = base + i * _CHUNK
            pltpu.sync_copy(idx_hbm.at[pl.ds(off, _CHUNK)], idx_v)
            pltpu.async_copy(tab_hbm.at[idx_v], rows_v, sem).wait()
            pltpu.sync_copy(rows_v, out_hbm.at[pl.ds(off, _CHUNK)])
            return 0

        lax.fori_loop(0, per_w // _CHUNK, chunk, 0)

    return run


def kernel(xyz, points):
    x = xyz[:, :, 0]
    y = xyz[:, :, 1]
    z = xyz[:, :, 2]
    qx, qy, qz, fpsg = _run_fps(x, y, z)

    qxr = qx.reshape(_NBLK, _QB, 1)
    qyr = qy.reshape(_NBLK, _QB, 1)
    qzr = qz.reshape(_NBLK, _QB, 1)
    x3 = x.reshape(_B, 1, _N)
    y3 = y.reshape(_B, 1, _N)
    z3 = z.reshape(_B, 1, _N)
    p8 = jnp.concatenate([xyz.transpose(0, 2, 1),
                          jnp.zeros((_B, 5, _N), jnp.float32)], axis=1)
    knng = _run_knn(x3, y3, z3, p8, qxr, qyr, qzr)

    table = jnp.pad(jnp.concatenate([points, xyz], axis=-1),
                    ((0, 0), (0, 0), (0, 61))).reshape(_B * _N, 128)

    new_rows = _gather_rows(_B * _G)(table, fpsg.reshape(-1))
    grouped_rows = _gather_rows(_B * _G * _K)(table, knng.reshape(-1))

    new_xyz = jnp.stack([qx, qy, qz], axis=-1)
    new_points = new_rows[:, :64].reshape(_B, _G, 64)
    grouped_xyz = grouped_rows[:, 64:67].reshape(_B, _G, _K, 3)
    grouped_points = grouped_rows[:, :64].reshape(_B, _G, _K, 64)
    return (new_xyz, new_points, grouped_xyz, grouped_points)
